# baseline (speedup 1.0000x reference)
import functools

import jax
import jax.numpy as jnp
from jax.experimental import pallas as pl
from jax.experimental.pallas import tpu as pltpu


def _mlp_body(x_ref, w1t_ref, b1_ref, w2t_ref, b2_ref, o_ref):
    x = x_ref[...]
    h = jnp.dot(w1t_ref[...], x, preferred_element_type=jnp.float32)
    h = jnp.maximum(h + b1_ref[...], 0.0)
    y = jnp.dot(w2t_ref[...], h, preferred_element_type=jnp.float32)
    o_ref[...] = jnp.maximum(y + b2_ref[...], 0.0)


@functools.partial(jax.jit, static_argnames=("block_cols",))
def _run(xt, w1t, b1c, w2t, b2c, block_cols):
    d, e = xt.shape
    grid = e // block_cols
    return pl.pallas_call(
        _mlp_body,
        grid=(grid,),
        in_specs=[
            pl.BlockSpec((d, block_cols), lambda i: (0, i)),
            pl.BlockSpec((d, d), lambda i: (0, 0)),
            pl.BlockSpec((d, 1), lambda i: (0, 0)),
            pl.BlockSpec((d, d), lambda i: (0, 0)),
            pl.BlockSpec((d, 1), lambda i: (0, 0)),
        ],
        out_specs=pl.BlockSpec((d, block_cols), lambda i: (0, i)),
        out_shape=jax.ShapeDtypeStruct((d, e), jnp.float32),
        compiler_params=pltpu.CompilerParams(
            dimension_semantics=("parallel",),
        ),
    )(xt, w1t, b1c, w2t, b2c)


def kernel(edge_attr, W1, b1, W2, b2):
    e, d = edge_attr.shape
    xt = edge_attr.T
    w1t = W1.astype(jnp.float32).T
    w2t = W2.astype(jnp.float32).T
    b1c = b1.astype(jnp.float32).reshape(d, 1)
    b2c = b2.astype(jnp.float32).reshape(d, 1)
    block_cols = next(bc for bc in (128000, 64000, 32000, 16000, 8000, 4000,
                                    2000, 1000, 128)
                      if e % bc == 0)
    out_t = _run(xt, w1t, b1c, w2t, b2c, block_cols=block_cols)
    return out_t.T


# speedup vs baseline: 1.0189x; 1.0189x over previous
import functools

import jax
import jax.numpy as jnp
from jax.experimental import pallas as pl
from jax.experimental.pallas import tpu as pltpu


def _outer(x_hbm, w1t_ref, b1_ref, w2t_ref, b2_ref, o_hbm,
           xbuf, ybuf, insem, outsem, *, block_cols):
    d, e = x_hbm.shape
    bc = block_cols
    g = e // bc
    w1t = w1t_ref[...]
    b1 = b1_ref[...]
    w2t = w2t_ref[...]
    b2 = b2_ref[...]

    def in_copy(i, slot):
        return pltpu.make_async_copy(
            x_hbm.at[:, pl.ds(i * bc, bc)], xbuf.at[slot], insem.at[slot])

    def out_copy(i, slot):
        return pltpu.make_async_copy(
            ybuf.at[slot], o_hbm.at[:, pl.ds(i * bc, bc)], outsem.at[slot])

    in_copy(0, 0).start()
    in_copy(1, 1).start()

    def body(i, carry):
        slot = jax.lax.rem(i, 2)
        in_copy(i, slot).wait()

        @pl.when(i >= 2)
        def _():
            out_copy(i - 2, slot).wait()

        x = xbuf[slot]
        h = jnp.dot(w1t, x, preferred_element_type=jnp.float32)
        h = jnp.maximum(h + b1, 0.0)
        y = jnp.dot(w2t, h, preferred_element_type=jnp.float32)
        ybuf[slot] = jnp.maximum(y + b2, 0.0)
        out_copy(i, slot).start()

        @pl.when(i + 2 < g)
        def _():
            in_copy(i + 2, slot).start()

        return carry

    jax.lax.fori_loop(0, g, body, 0)
    out_copy(g - 2, jax.lax.rem(g - 2, 2)).wait()
    out_copy(g - 1, jax.lax.rem(g - 1, 2)).wait()


@functools.partial(jax.jit, static_argnames=("block_cols",))
def _run(xt, w1t, b1c, w2t, b2c, block_cols):
    d, e = xt.shape
    return pl.pallas_call(
        functools.partial(_outer, block_cols=block_cols),
        in_specs=[
            pl.BlockSpec(memory_space=pltpu.MemorySpace.HBM),
            pl.BlockSpec(memory_space=pltpu.MemorySpace.VMEM),
            pl.BlockSpec(memory_space=pltpu.MemorySpace.VMEM),
            pl.BlockSpec(memory_space=pltpu.MemorySpace.VMEM),
            pl.BlockSpec(memory_space=pltpu.MemorySpace.VMEM),
        ],
        out_specs=pl.BlockSpec(memory_space=pltpu.MemorySpace.HBM),
        out_shape=jax.ShapeDtypeStruct((d, e), jnp.float32),
        scratch_shapes=[
            pltpu.VMEM((2, d, block_cols), jnp.float32),
            pltpu.VMEM((2, d, block_cols), jnp.float32),
            pltpu.SemaphoreType.DMA((2,)),
            pltpu.SemaphoreType.DMA((2,)),
        ],
    )(xt, w1t, b1c, w2t, b2c)


def kernel(edge_attr, W1, b1, W2, b2):
    e, d = edge_attr.shape
    xt = edge_attr.T
    w1t = W1.astype(jnp.float32).T
    w2t = W2.astype(jnp.float32).T
    b1c = b1.astype(jnp.float32).reshape(d, 1)
    b2c = b2.astype(jnp.float32).reshape(d, 1)
    block_cols = next(bc for bc in (160000, 64000, 32000, 16000, 8000, 4000,
                                    2000, 1000, 128)
                      if e % bc == 0)
    out_t = _run(xt, w1t, b1c, w2t, b2c, block_cols=block_cols)
    return out_t.T


# speedup vs baseline: 1.0192x; 1.0003x over previous
import functools

_SLOTS = 3

import jax
import jax.numpy as jnp
from jax.experimental import pallas as pl
from jax.experimental.pallas import tpu as pltpu


def _outer(x_hbm, w1t_ref, b1_ref, w2t_ref, b2_ref, o_hbm,
           xbuf, ybuf, insem, outsem, *, block_cols):
    d, e = x_hbm.shape
    bc = block_cols
    g = e // bc
    w1t = w1t_ref[...]
    b1 = b1_ref[...]
    w2t = w2t_ref[...]
    b2 = b2_ref[...]

    def in_copy(i, slot):
        return pltpu.make_async_copy(
            x_hbm.at[:, pl.ds(i * bc, bc)], xbuf.at[slot], insem.at[slot])

    def out_copy(i, slot):
        return pltpu.make_async_copy(
            ybuf.at[slot], o_hbm.at[:, pl.ds(i * bc, bc)], outsem.at[slot])

    for s in range(_SLOTS):
        in_copy(s, s).start()

    def body(i, carry):
        slot = jax.lax.rem(i, _SLOTS)
        in_copy(i, slot).wait()

        @pl.when(i >= _SLOTS)
        def _():
            out_copy(i - _SLOTS, slot).wait()

        x = xbuf[slot]
        h = jnp.dot(w1t, x, preferred_element_type=jnp.float32)
        h = jnp.maximum(h + b1, 0.0)
        y = jnp.dot(w2t, h, preferred_element_type=jnp.float32)
        ybuf[slot] = jnp.maximum(y + b2, 0.0)
        out_copy(i, slot).start()

        @pl.when(i + _SLOTS < g)
        def _():
            in_copy(i + _SLOTS, slot).start()

        return carry

    jax.lax.fori_loop(0, g, body, 0)
    for s in range(_SLOTS):
        j = g - _SLOTS + s
        out_copy(j, j % _SLOTS).wait()


@functools.partial(jax.jit, static_argnames=("block_cols",))
def _run(xt, w1t, b1c, w2t, b2c, block_cols):
    d, e = xt.shape
    return pl.pallas_call(
        functools.partial(_outer, block_cols=block_cols),
        in_specs=[
            pl.BlockSpec(memory_space=pltpu.MemorySpace.HBM),
            pl.BlockSpec(memory_space=pltpu.MemorySpace.VMEM),
            pl.BlockSpec(memory_space=pltpu.MemorySpace.VMEM),
            pl.BlockSpec(memory_space=pltpu.MemorySpace.VMEM),
            pl.BlockSpec(memory_space=pltpu.MemorySpace.VMEM),
        ],
        out_specs=pl.BlockSpec(memory_space=pltpu.MemorySpace.HBM),
        out_shape=jax.ShapeDtypeStruct((d, e), jnp.float32),
        scratch_shapes=[
            pltpu.VMEM((_SLOTS, d, block_cols), jnp.float32),
            pltpu.VMEM((_SLOTS, d, block_cols), jnp.float32),
            pltpu.SemaphoreType.DMA((_SLOTS,)),
            pltpu.SemaphoreType.DMA((_SLOTS,)),
        ],
    )(xt, w1t, b1c, w2t, b2c)


def kernel(edge_attr, W1, b1, W2, b2):
    e, d = edge_attr.shape
    xt = edge_attr.T
    w1t = W1.astype(jnp.float32).T
    w2t = W2.astype(jnp.float32).T
    b1c = b1.astype(jnp.float32).reshape(d, 1)
    b2c = b2.astype(jnp.float32).reshape(d, 1)
    block_cols = next(bc for bc in (128000, 64000, 32000, 16000, 8000, 4000,
                                    2000, 1000, 128)
                      if e % bc == 0)
    out_t = _run(xt, w1t, b1c, w2t, b2c, block_cols=block_cols)
    return out_t.T
